# Initial kernel scaffold; baseline (speedup 1.0000x reference)
import jax
import jax.numpy as jnp
from jax.experimental import pallas as pl


def kernel(feat, conv_w, conv_b, w_obj, b_obj, w_reg, b_reg):
    raise NotImplementedError("write your pallas kernel here")


# speedup vs baseline: 1.5798x; 1.5798x over previous
import jax
import jax.numpy as jnp
from jax.experimental import pallas as pl

_ANCHOR_LENGTHS = (1.0, 2.0, 3.0, 4.0, 5.0, 7.0, 9.0)
_A = len(_ANCHOR_LENGTHS)


def _anchors_1d(Lf):
    lengths = jnp.array(_ANCHOR_LENGTHS, dtype=jnp.float32)
    centers = jnp.arange(Lf, dtype=jnp.float32) + 0.5
    c = jnp.broadcast_to(centers[:, None], (Lf, _A))
    w = jnp.broadcast_to(lengths[None, :], (Lf, _A))
    return jnp.stack([c - 0.5 * w, c + 0.5 * w], axis=-1).reshape(Lf * _A, 2)


def _rpn_kernel(f_ref, wt_ref, cb_ref, wh_ref, bh_ref, out_ref):
    f = f_ref[0]
    C, L = f.shape
    g0 = jax.lax.dot(wt_ref[1], f, preferred_element_type=jnp.float32)
    gm = jax.lax.dot(wt_ref[0], f, preferred_element_type=jnp.float32)
    gp = jax.lax.dot(wt_ref[2], f, preferred_element_type=jnp.float32)
    zero_col = jnp.zeros((C, 1), dtype=jnp.float32)
    h = g0
    h = h + jnp.concatenate([zero_col, gm[:, :-1]], axis=1)
    h = h + jnp.concatenate([gp[:, 1:], zero_col], axis=1)
    h = jnp.maximum(h + cb_ref[...], 0.0)
    out = jax.lax.dot(wh_ref[...], h, preferred_element_type=jnp.float32)
    out_ref[0] = out + bh_ref[...]


def kernel(feat, conv_w, conv_b, w_obj, b_obj, w_reg, b_reg):
    B, C, Lf = feat.shape
    H = w_obj.shape[0] + w_reg.shape[0]
    w_taps = jnp.transpose(conv_w, (2, 0, 1))
    wh = jnp.concatenate([w_obj, w_reg], axis=0)
    bh = jnp.concatenate([b_obj, b_reg])[:, None]
    cb = conv_b[:, None]
    out = pl.pallas_call(
        _rpn_kernel,
        grid=(B,),
        in_specs=[
            pl.BlockSpec((1, C, Lf), lambda b: (b, 0, 0)),
            pl.BlockSpec((3, C, C), lambda b: (0, 0, 0)),
            pl.BlockSpec((C, 1), lambda b: (0, 0)),
            pl.BlockSpec((H, C), lambda b: (0, 0)),
            pl.BlockSpec((H, 1), lambda b: (0, 0)),
        ],
        out_specs=pl.BlockSpec((1, H, Lf), lambda b: (b, 0, 0)),
        out_shape=jax.ShapeDtypeStruct((B, H, Lf), jnp.float32),
    )(feat, w_taps, cb, wh, bh)
    obj = jnp.transpose(out[:, :_A], (0, 2, 1)).reshape(B, Lf * _A)
    reg = jnp.transpose(out[:, _A:], (0, 2, 1)).reshape(B, Lf * _A, 2)
    return obj, reg, _anchors_1d(Lf)
